# baseline (speedup 1.0000x reference)
import jax
import jax.numpy as jnp
import numpy as np
from jax import lax
from jax.experimental import pallas as pl
from jax.experimental.pallas import tpu as pltpu
from jax.experimental.pallas import tpu_sc as plsc

NUM_INPUT = 4096
HIDDEN_BATCHES = 23
HIDDEN_SIZE = 4096
NUM_OUTPUT = 1024
FAN_IN = 32
LANES = 16
NUM_TILES = 16
ROWS_HID = HIDDEN_SIZE // NUM_TILES
ROWS_OUT = NUM_OUTPUT // NUM_TILES
PACK_MINOR = 128
HID_PROWS = HIDDEN_SIZE * FAN_IN // PACK_MINOR
OUT_PROWS = NUM_OUTPUT * FAN_IN // PACK_MINOR
HID_PCHUNK = HID_PROWS // NUM_TILES
OUT_PCHUNK = OUT_PROWS // NUM_TILES
MASK_HI = np.int32(-65536)
MASK_LO = np.int32(65535)


def _rows16(cbuf, vals, bias_vec, row0, slot=None):
    prods = []
    for i in range(LANES):
        r = row0 + i
        if slot is None:
            c0 = cbuf[r, pl.ds(0, LANES)]
            c1 = cbuf[r, pl.ds(LANES, LANES)]
        else:
            c0 = cbuf[slot, r, pl.ds(0, LANES)]
            c1 = cbuf[slot, r, pl.ds(LANES, LANES)]
        w0 = plsc.bitcast(c0 & MASK_HI, jnp.float32)
        w1 = plsc.bitcast(c1 & MASK_HI, jnp.float32)
        g0 = plsc.load_gather(vals, [c0 & MASK_LO])
        g1 = plsc.load_gather(vals, [c1 & MASK_LO])
        prods.append(w0 * g0 + w1 * g1)
    lane = lax.iota(jnp.int32, LANES)
    d = 1
    while len(prods) > 1:
        pidx = lane ^ d
        m = (lane & d) == 0
        nxt = []
        for k in range(0, len(prods), 2):
            a, b = prods[k], prods[k + 1]
            pa = jnp.take_along_axis(a, pidx, axis=0)
            pb = jnp.take_along_axis(b, pidx, axis=0)
            nxt.append(jnp.where(m, a, pb) + jnp.where(m, pa, b))
        prods = nxt
        d *= 2
    return prods[0] + bias_vec


def _body(x_hbm, ch_hbm, co_hbm, bias_hbm, out_hbm,
          vals, cbuf2, cobuf, bbuf2, obbuf, obuf, shared,
          csem, bsem, osem):
    cid = lax.axis_index("c")
    sid = lax.axis_index("s")

    @pl.when(cid == 0)
    def _():
        base = sid * ROWS_HID
        base_o = sid * ROWS_OUT
        def issue(t, slot):
            pltpu.async_copy(ch_hbm.at[pl.ds(t, 1), pl.ds(base, ROWS_HID), :],
                             cbuf2.at[pl.ds(slot, 1)], csem)
            pltpu.async_copy(bias_hbm.at[pl.ds(t * HIDDEN_SIZE + base, ROWS_HID)],
                             bbuf2.at[pl.ds(slot * ROWS_HID, ROWS_HID)], bsem)

        def wait(t, slot):
            pltpu.make_async_copy(ch_hbm.at[pl.ds(t, 1), pl.ds(base, ROWS_HID), :],
                                  cbuf2.at[pl.ds(slot, 1)], csem).wait()
            pltpu.make_async_copy(bias_hbm.at[pl.ds(t * HIDDEN_SIZE + base, ROWS_HID)],
                                  bbuf2.at[pl.ds(slot * ROWS_HID, ROWS_HID)], bsem).wait()

        issue(0, 0)
        pltpu.async_copy(co_hbm.at[pl.ds(base_o, ROWS_OUT), :], cobuf, osem)
        pltpu.async_copy(
            bias_hbm.at[pl.ds(HIDDEN_BATCHES * HIDDEN_SIZE + base_o, ROWS_OUT)],
            obbuf, osem)
        pltpu.sync_copy(x_hbm, vals)

        def layer(t, carry):
            slot = lax.rem(t, 2)
            wait(t, slot)

            @pl.when(t + 1 < HIDDEN_BATCHES)
            def _():
                issue(t + 1, lax.rem(t + 1, 2))

            def rows(r, c2):
                row0 = r * LANES
                bv = bbuf2[pl.ds(slot * ROWS_HID + row0, LANES)]
                a = _rows16(cbuf2, vals, bv, row0, slot=slot)
                obuf[pl.ds(row0, LANES)] = a / (1.0 + jnp.exp(-a))
                return c2

            lax.fori_loop(0, ROWS_HID // LANES, rows, 0)

            pltpu.sync_copy(obuf, shared.at[slot, pl.ds(base, ROWS_HID)])
            plsc.subcore_barrier()
            pltpu.sync_copy(shared.at[slot], vals)
            return carry

        lax.fori_loop(0, HIDDEN_BATCHES, layer, 0)

        pltpu.make_async_copy(co_hbm.at[pl.ds(base_o, ROWS_OUT), :], cobuf, osem).wait()
        pltpu.make_async_copy(
            bias_hbm.at[pl.ds(HIDDEN_BATCHES * HIDDEN_SIZE + base_o, ROWS_OUT)],
            obbuf, osem).wait()

        def out_rows(r, c2):
            row0 = r * LANES
            bv = obbuf[pl.ds(row0, LANES)]
            obuf[pl.ds(row0, LANES)] = _rows16(cobuf, vals, bv, row0)
            return c2

        lax.fori_loop(0, ROWS_OUT // LANES, out_rows, 0)
        pltpu.sync_copy(obuf.at[pl.ds(0, ROWS_OUT)], out_hbm.at[pl.ds(base_o, ROWS_OUT)])


def _pack(weights, idx, local_start):
    wbits = lax.bitcast_convert_type(
        weights.astype(jnp.bfloat16), jnp.uint16).astype(jnp.uint32)
    ibits = (idx - local_start).astype(jnp.uint32)
    return lax.bitcast_convert_type((wbits << 16) | ibits, jnp.int32)


def kernel(x, hidden_weights, out_weights, bias, hidden_idx, out_idx):
    pstart = (np.arange(HIDDEN_BATCHES, dtype=np.int32)
              * HIDDEN_SIZE)[:, None, None]
    ch = _pack(hidden_weights, hidden_idx, pstart)
    co = _pack(out_weights, out_idx, HIDDEN_BATCHES * HIDDEN_SIZE)

    mesh = plsc.VectorSubcoreMesh(core_axis_name="c", subcore_axis_name="s")
    run = pl.kernel(
        _body,
        mesh=mesh,
        compiler_params=pltpu.CompilerParams(needs_layout_passes=False),
        out_type=jax.ShapeDtypeStruct((NUM_OUTPUT,), jnp.float32),
        scratch_types=[
            pltpu.VMEM((HIDDEN_SIZE,), jnp.float32),
            pltpu.VMEM((2, ROWS_HID, FAN_IN), jnp.int32),
            pltpu.VMEM((ROWS_OUT, FAN_IN), jnp.int32),
            pltpu.VMEM((2 * ROWS_HID,), jnp.float32),
            pltpu.VMEM((ROWS_OUT,), jnp.float32),
            pltpu.VMEM((ROWS_HID,), jnp.float32),
            pltpu.VMEM_SHARED((2, HIDDEN_SIZE), jnp.float32),
            pltpu.SemaphoreType.DMA,
            pltpu.SemaphoreType.DMA,
            pltpu.SemaphoreType.DMA,
        ],
    )
    return run(x, ch, co, bias)


# speedup vs baseline: 1.1357x; 1.1357x over previous
import jax
import jax.numpy as jnp
import numpy as np
from jax import lax
from jax.experimental import pallas as pl
from jax.experimental.pallas import tpu as pltpu
from jax.experimental.pallas import tpu_sc as plsc

NUM_INPUT = 4096
HIDDEN_BATCHES = 23
HIDDEN_SIZE = 4096
NUM_OUTPUT = 1024
FAN_IN = 32
LANES = 16
NUM_TILES = 16
ROWS_HID = HIDDEN_SIZE // NUM_TILES
ROWS_OUT = NUM_OUTPUT // NUM_TILES
PACK_MINOR = 128
HID_PROWS = HIDDEN_SIZE * FAN_IN // PACK_MINOR
OUT_PROWS = NUM_OUTPUT * FAN_IN // PACK_MINOR
HID_PCHUNK = HID_PROWS // NUM_TILES
OUT_PCHUNK = OUT_PROWS // NUM_TILES
MASK_HI = np.int32(-65536)
MASK_LO = np.int32(65535)


def _rows16(cbuf, vals, bias_vec, base_off):
    prods = []
    for i in range(LANES):
        off = base_off + i * FAN_IN
        c0 = cbuf[pl.ds(off, LANES)]
        c1 = cbuf[pl.ds(off + LANES, LANES)]
        w0 = plsc.bitcast(c0 & MASK_HI, jnp.float32)
        w1 = plsc.bitcast(c1 & MASK_HI, jnp.float32)
        g0 = plsc.load_gather(vals, [c0 & MASK_LO])
        g1 = plsc.load_gather(vals, [c1 & MASK_LO])
        prods.append(w0 * g0 + w1 * g1)
    lane = lax.iota(jnp.int32, LANES)
    d = 1
    while len(prods) > 1:
        pidx = lane ^ d
        m = (lane & d) == 0
        nxt = []
        for k in range(0, len(prods), 2):
            a, b = prods[k], prods[k + 1]
            pa = jnp.take_along_axis(a, pidx, axis=0)
            pb = jnp.take_along_axis(b, pidx, axis=0)
            nxt.append(jnp.where(m, a, pb) + jnp.where(m, pa, b))
        prods = nxt
        d *= 2
    return prods[0] + bias_vec


def _body(x_hbm, ch_hbm, co_hbm, bias_hbm, out_hbm,
          vals, cbuf2, cobuf, bbuf2, obbuf, obuf, shared,
          csem, bsem, osem):
    cid = lax.axis_index("c")
    sid = lax.axis_index("s")

    @pl.when(cid == 0)
    def _():
        base = sid * ROWS_HID
        base_o = sid * ROWS_OUT
        cw = ROWS_HID * FAN_IN
        ocw = ROWS_OUT * FAN_IN

        def issue(t, slot):
            pltpu.async_copy(
                ch_hbm.at[pl.ds(t * (HIDDEN_SIZE * FAN_IN) + sid * cw, cw)],
                cbuf2.at[pl.ds(slot * cw, cw)], csem)
            pltpu.async_copy(bias_hbm.at[pl.ds(t * HIDDEN_SIZE + base, ROWS_HID)],
                             bbuf2.at[pl.ds(slot * ROWS_HID, ROWS_HID)], bsem)

        def wait(t, slot):
            pltpu.make_async_copy(
                ch_hbm.at[pl.ds(t * (HIDDEN_SIZE * FAN_IN) + sid * cw, cw)],
                cbuf2.at[pl.ds(slot * cw, cw)], csem).wait()
            pltpu.make_async_copy(bias_hbm.at[pl.ds(t * HIDDEN_SIZE + base, ROWS_HID)],
                                  bbuf2.at[pl.ds(slot * ROWS_HID, ROWS_HID)], bsem).wait()

        issue(0, 0)
        pltpu.async_copy(co_hbm.at[pl.ds(sid * ocw, ocw)], cobuf, osem)
        pltpu.async_copy(
            bias_hbm.at[pl.ds(HIDDEN_BATCHES * HIDDEN_SIZE + base_o, ROWS_OUT)],
            obbuf, osem)
        pltpu.sync_copy(x_hbm, vals)

        def layer(t, carry):
            slot = lax.rem(t, 2)
            wait(t, slot)

            @pl.when(t + 1 < HIDDEN_BATCHES)
            def _():
                issue(t + 1, lax.rem(t + 1, 2))

            def rows(r, c2):
                row0 = r * LANES
                bv = bbuf2[pl.ds(slot * ROWS_HID + row0, LANES)]
                a = _rows16(cbuf2, vals, bv,
                            slot * (ROWS_HID * FAN_IN) + row0 * FAN_IN)
                obuf[pl.ds(row0, LANES)] = a / (1.0 + jnp.exp(-a))
                return c2

            lax.fori_loop(0, ROWS_HID // LANES, rows, 0)

            pltpu.sync_copy(obuf, shared.at[slot, pl.ds(base, ROWS_HID)])
            plsc.subcore_barrier()
            pltpu.sync_copy(shared.at[slot], vals)
            return carry

        lax.fori_loop(0, HIDDEN_BATCHES, layer, 0)

        pltpu.make_async_copy(co_hbm.at[pl.ds(sid * ocw, ocw)], cobuf, osem).wait()
        pltpu.make_async_copy(
            bias_hbm.at[pl.ds(HIDDEN_BATCHES * HIDDEN_SIZE + base_o, ROWS_OUT)],
            obbuf, osem).wait()

        def out_rows(r, c2):
            row0 = r * LANES
            bv = obbuf[pl.ds(row0, LANES)]
            obuf[pl.ds(row0, LANES)] = _rows16(cobuf, vals, bv, row0 * FAN_IN)
            return c2

        lax.fori_loop(0, ROWS_OUT // LANES, out_rows, 0)
        pltpu.sync_copy(obuf.at[pl.ds(0, ROWS_OUT)], out_hbm.at[pl.ds(base_o, ROWS_OUT)])


def _pack(weights, idx, local_start):
    wbits = lax.bitcast_convert_type(
        weights.astype(jnp.bfloat16), jnp.uint16).astype(jnp.uint32)
    ibits = (idx - local_start).astype(jnp.uint32)
    packed = lax.bitcast_convert_type((wbits << 16) | ibits, jnp.int32)
    return packed.reshape(-1)


def kernel(x, hidden_weights, out_weights, bias, hidden_idx, out_idx):
    pstart = (np.arange(HIDDEN_BATCHES, dtype=np.int32)
              * HIDDEN_SIZE)[:, None, None]
    ch = _pack(hidden_weights, hidden_idx, pstart)
    co = _pack(out_weights, out_idx, HIDDEN_BATCHES * HIDDEN_SIZE)

    mesh = plsc.VectorSubcoreMesh(core_axis_name="c", subcore_axis_name="s")
    run = pl.kernel(
        _body,
        mesh=mesh,
        compiler_params=pltpu.CompilerParams(needs_layout_passes=False),
        out_type=jax.ShapeDtypeStruct((NUM_OUTPUT,), jnp.float32),
        scratch_types=[
            pltpu.VMEM((HIDDEN_SIZE,), jnp.float32),
            pltpu.VMEM((2 * ROWS_HID * FAN_IN,), jnp.int32),
            pltpu.VMEM((ROWS_OUT * FAN_IN,), jnp.int32),
            pltpu.VMEM((2 * ROWS_HID,), jnp.float32),
            pltpu.VMEM((ROWS_OUT,), jnp.float32),
            pltpu.VMEM((ROWS_HID,), jnp.float32),
            pltpu.VMEM_SHARED((2, HIDDEN_SIZE), jnp.float32),
            pltpu.SemaphoreType.DMA,
            pltpu.SemaphoreType.DMA,
            pltpu.SemaphoreType.DMA,
        ],
    )
    return run(x, ch, co, bias)


# speedup vs baseline: 1.1441x; 1.0073x over previous
import jax
import jax.numpy as jnp
import numpy as np
from jax import lax
from jax.experimental import pallas as pl
from jax.experimental.pallas import tpu as pltpu
from jax.experimental.pallas import tpu_sc as plsc

NUM_INPUT = 4096
HIDDEN_BATCHES = 23
HIDDEN_SIZE = 4096
NUM_OUTPUT = 1024
FAN_IN = 32
LANES = 16
NUM_TILES = 16
ROWS_HID = HIDDEN_SIZE // NUM_TILES
ROWS_OUT = NUM_OUTPUT // NUM_TILES
MASK_HI = np.int32(-65536)
MASK_LO = np.int32(65535)


def _rows16(cbuf, vals, bias_vec, base_off):
    prods = []
    for i in range(LANES):
        off = base_off + i * FAN_IN
        c0 = cbuf[pl.ds(off, LANES)]
        c1 = cbuf[pl.ds(off + LANES, LANES)]
        w0 = plsc.bitcast(c0 & MASK_HI, jnp.float32)
        w1 = plsc.bitcast(c1 & MASK_HI, jnp.float32)
        g0 = plsc.load_gather(vals, [c0 & MASK_LO])
        g1 = plsc.load_gather(vals, [c1 & MASK_LO])
        prods.append(w0 * g0 + w1 * g1)
    lane = lax.iota(jnp.int32, LANES)
    d = 1
    while len(prods) > 1:
        pidx = lane ^ d
        m = (lane & d) == 0
        nxt = []
        for k in range(0, len(prods), 2):
            a, b = prods[k], prods[k + 1]
            pa = jnp.take_along_axis(a, pidx, axis=0)
            pb = jnp.take_along_axis(b, pidx, axis=0)
            nxt.append(jnp.where(m, a, pb) + jnp.where(m, pa, b))
        prods = nxt
        d *= 2
    return prods[0] + bias_vec


def _body(x_hbm, ch_hbm, co_hbm, bias_hbm, out_hbm,
          vals, cbuf2, cobuf, bbuf2, obbuf, obuf, shared,
          csem, bsem, osem):
    cid = lax.axis_index("c")
    sid = lax.axis_index("s")

    @pl.when(cid == 0)
    def _():
        base = sid * ROWS_HID
        base_o = sid * ROWS_OUT
        cw = ROWS_HID * FAN_IN
        ocw = ROWS_OUT * FAN_IN

        def issue(t, slot):
            pltpu.async_copy(
                ch_hbm.at[pl.ds(t * (HIDDEN_SIZE * FAN_IN) + sid * cw, cw)],
                cbuf2.at[pl.ds(slot * cw, cw)], csem)
            pltpu.async_copy(bias_hbm.at[pl.ds(t * HIDDEN_SIZE + base, ROWS_HID)],
                             bbuf2.at[pl.ds(slot * ROWS_HID, ROWS_HID)], bsem)

        def wait(t, slot):
            pltpu.make_async_copy(
                ch_hbm.at[pl.ds(t * (HIDDEN_SIZE * FAN_IN) + sid * cw, cw)],
                cbuf2.at[pl.ds(slot * cw, cw)], csem).wait()
            pltpu.make_async_copy(bias_hbm.at[pl.ds(t * HIDDEN_SIZE + base, ROWS_HID)],
                                  bbuf2.at[pl.ds(slot * ROWS_HID, ROWS_HID)], bsem).wait()

        issue(0, 0)
        pltpu.async_copy(co_hbm.at[pl.ds(sid * ocw, ocw)], cobuf, osem)
        pltpu.async_copy(
            bias_hbm.at[pl.ds(HIDDEN_BATCHES * HIDDEN_SIZE + base_o, ROWS_OUT)],
            obbuf, osem)
        pltpu.sync_copy(x_hbm, vals)

        def layer(t, carry):
            slot = lax.rem(t, 2)
            wait(t, slot)

            @pl.when(t + 1 < HIDDEN_BATCHES)
            def _():
                issue(t + 1, lax.rem(t + 1, 2))

            def rows(r, c2):
                row0 = r * LANES
                bv = bbuf2[pl.ds(slot * ROWS_HID + row0, LANES)]
                a = _rows16(cbuf2, vals, bv,
                            slot * (ROWS_HID * FAN_IN) + row0 * FAN_IN)
                obuf[pl.ds(row0, LANES)] = a / (1.0 + jnp.exp(-a))
                return c2

            lax.fori_loop(0, ROWS_HID // LANES, rows, 0)

            pltpu.sync_copy(obuf, shared.at[slot, pl.ds(base, ROWS_HID)])
            plsc.subcore_barrier()
            pltpu.sync_copy(shared.at[slot], vals)
            return carry

        lax.fori_loop(0, HIDDEN_BATCHES, layer, 0)

        pltpu.make_async_copy(co_hbm.at[pl.ds(sid * ocw, ocw)], cobuf, osem).wait()
        pltpu.make_async_copy(
            bias_hbm.at[pl.ds(HIDDEN_BATCHES * HIDDEN_SIZE + base_o, ROWS_OUT)],
            obbuf, osem).wait()

        def out_rows(r, c2):
            row0 = r * LANES
            bv = obbuf[pl.ds(row0, LANES)]
            obuf[pl.ds(row0, LANES)] = _rows16(cobuf, vals, bv, row0 * FAN_IN)
            return c2

        lax.fori_loop(0, ROWS_OUT // LANES, out_rows, 0)
        pltpu.sync_copy(obuf.at[pl.ds(0, ROWS_OUT)], out_hbm.at[pl.ds(base_o, ROWS_OUT)])


def _pack(weights, idx, local_start):
    wbits = lax.bitcast_convert_type(
        weights.astype(jnp.bfloat16), jnp.uint16).astype(jnp.uint32)
    ibits = (idx - local_start).astype(jnp.uint32)
    packed = lax.bitcast_convert_type((wbits << 16) | ibits, jnp.int32)
    return packed.reshape(-1)


def kernel(x, hidden_weights, out_weights, bias, hidden_idx, out_idx):
    pstart = (np.arange(HIDDEN_BATCHES, dtype=np.int32)
              * HIDDEN_SIZE)[:, None, None]
    ch = _pack(hidden_weights, hidden_idx, pstart)
    co = _pack(out_weights, out_idx, HIDDEN_BATCHES * HIDDEN_SIZE)

    mesh = plsc.VectorSubcoreMesh(core_axis_name="c", subcore_axis_name="s")
    run = pl.kernel(
        _body,
        mesh=mesh,
        compiler_params=pltpu.CompilerParams(needs_layout_passes=False),
        out_type=jax.ShapeDtypeStruct((NUM_OUTPUT,), jnp.float32),
        scratch_types=[
            pltpu.VMEM((HIDDEN_SIZE,), jnp.float32),
            pltpu.VMEM((2 * ROWS_HID * FAN_IN,), jnp.int32),
            pltpu.VMEM((ROWS_OUT * FAN_IN,), jnp.int32),
            pltpu.VMEM((2 * ROWS_HID,), jnp.float32),
            pltpu.VMEM((ROWS_OUT,), jnp.float32),
            pltpu.VMEM((ROWS_HID,), jnp.float32),
            pltpu.VMEM_SHARED((2, HIDDEN_SIZE), jnp.float32),
            pltpu.SemaphoreType.DMA,
            pltpu.SemaphoreType.DMA,
            pltpu.SemaphoreType.DMA,
        ],
    )
    return run(x, ch, co, bias)
